# baseline (speedup 1.0000x reference)
import jax
import jax.numpy as jnp
from jax.experimental import pallas as pl

_K = 5


def _body(wt_ref, *refs):
    xrefs = refs[:_K]
    o_ref = refs[_K]
    w = wt_ref[...]
    for k in range(_K):
        o_ref[k] = jax.lax.dot_general(
            w, xrefs[k][0], (((1,), (0,)), ((), ())),
            preferred_element_type=jnp.float32)


def kernel(x, idx, vals, W):
    B, N, C = x.shape
    F = W.shape[1]
    xt = jnp.transpose(x, (1, 2, 0))
    wt = W.T

    def xmap(k):
        return lambda i: (_K * i + k, 0, 0)

    out_t = pl.pallas_call(
        _body,
        grid=(N // _K,),
        in_specs=[pl.BlockSpec((F, C), lambda i: (0, 0))] +
                 [pl.BlockSpec((1, C, B), xmap(k)) for k in range(_K)],
        out_specs=pl.BlockSpec((_K, F, B), lambda i: (i, 0, 0)),
        out_shape=jax.ShapeDtypeStruct((N, F, B), x.dtype),
    )(wt, *([xt] * _K))
    return jnp.transpose(out_t, (2, 1, 0))


# speedup vs baseline: 1.0199x; 1.0199x over previous
import jax
import jax.numpy as jnp
from jax.experimental import pallas as pl

_K = 4
_NB = 2
_HB = 2


def _body(wt_ref, *refs):
    xrefs = refs[:_K]
    o_ref = refs[_K]
    w = wt_ref[...]
    hb = xrefs[0].shape[2]
    for k in range(_K):
        o_ref[k // _HB, :, pl.ds((k % _HB) * hb, hb)] = jax.lax.dot_general(
            w, xrefs[k][0], (((1,), (0,)), ((), ())),
            preferred_element_type=jnp.float32)


def kernel(x, idx, vals, W):
    B, N, C = x.shape
    F = W.shape[1]
    xt = jnp.transpose(x, (1, 2, 0))
    wt = W.T

    def xmap(k):
        return lambda i: (_NB * i + k // _HB, 0, k % _HB)

    out_t = pl.pallas_call(
        _body,
        grid=(N // _NB,),
        in_specs=[pl.BlockSpec((F, C), lambda i: (0, 0))] +
                 [pl.BlockSpec((1, C, B // _HB), xmap(k)) for k in range(_K)],
        out_specs=pl.BlockSpec((_NB, F, B), lambda i: (i, 0, 0)),
        out_shape=jax.ShapeDtypeStruct((N, F, B), x.dtype),
    )(wt, *([xt] * _K))
    return jnp.transpose(out_t, (2, 1, 0))
